# baseline (speedup 1.0000x reference)
import functools
import jax
import jax.numpy as jnp
from jax import lax
from jax.experimental import pallas as pl
from jax.experimental.pallas import tpu as pltpu
from jax.experimental.pallas import tpu_sc as plsc

N_PAD = 10240
HID = 256
BN = 2048
BE = 2000

NC, NS = 2, 16
NW = NC * NS
E_TOT = 320000
EW = E_TOT // NW
CG = 40
NCHG = EW // CG
CS = 32
NPART = 2 * NW
NPP = N_PAD // NPART


def _embed_body(h0_ref, w_ref, b_ref, out_ref):
    out_ref[...] = jnp.dot(h0_ref[...], w_ref[...],
                           preferred_element_type=jnp.float32) + b_ref[...]


def _embed(h0p, w, b):
    return pl.pallas_call(
        _embed_body,
        grid=(N_PAD // BN,),
        in_specs=[
            pl.BlockSpec((BN, 128), lambda i: (i, 0)),
            pl.BlockSpec((128, HID), lambda i: (0, 0)),
            pl.BlockSpec((1, HID), lambda i: (0, 0)),
        ],
        out_specs=pl.BlockSpec((BN, HID), lambda i: (i, 0)),
        out_shape=jax.ShapeDtypeStruct((N_PAD, HID), jnp.float32),
    )(h0p, w, b.reshape(1, HID))


def _nodeproj_body(h_ref, co_ref, wr_ref, wc_ref, b0_ref, tabr_ref, tabc_ref):
    h = h_ref[...]
    co = co_ref[...]
    tabr_ref[:, :HID] = jnp.dot(h, wr_ref[...],
                                preferred_element_type=jnp.float32) + b0_ref[...]
    tabr_ref[:, HID:] = co
    tabc_ref[:, :HID] = jnp.dot(h, wc_ref[...],
                                preferred_element_type=jnp.float32)
    tabc_ref[:, HID:] = co


def _nodeproj(h, coord, wr, wc, b0):
    return pl.pallas_call(
        _nodeproj_body,
        grid=(N_PAD // BN,),
        in_specs=[
            pl.BlockSpec((BN, HID), lambda i: (i, 0)),
            pl.BlockSpec((BN, 128), lambda i: (i, 0)),
            pl.BlockSpec((HID, HID), lambda i: (0, 0)),
            pl.BlockSpec((HID, HID), lambda i: (0, 0)),
            pl.BlockSpec((1, HID), lambda i: (0, 0)),
        ],
        out_specs=[
            pl.BlockSpec((BN, HID + 128), lambda i: (i, 0)),
            pl.BlockSpec((BN, HID + 128), lambda i: (i, 0)),
        ],
        out_shape=[
            jax.ShapeDtypeStruct((N_PAD, HID + 128), jnp.float32),
            jax.ShapeDtypeStruct((N_PAD, HID + 128), jnp.float32),
        ],
    )(h, coord, wr, wc, b0.reshape(1, HID))


def _edge_mlp_body(g_ref, ea_ref, w0e_ref, wrad_ref,
                   w1_ref, b1_ref, wc0_ref, bc0_ref, wc1_ref,
                   m2_ref, tr_ref):
    cdf = g_ref[:, HID:]
    radial = jnp.sum(cdf * cdf, axis=1, keepdims=True)
    pre = (g_ref[:, :HID] + radial * wrad_ref[...]
           + jnp.dot(ea_ref[...], w0e_ref[...],
                     preferred_element_type=jnp.float32))
    m = jax.nn.silu(pre)
    m2 = jax.nn.silu(jnp.dot(m, w1_ref[...],
                             preferred_element_type=jnp.float32) + b1_ref[...])
    p = jax.nn.silu(jnp.dot(m2, wc0_ref[...],
                            preferred_element_type=jnp.float32) + bc0_ref[...])
    s = jnp.sum(p * wc1_ref[...], axis=1, keepdims=True)
    m2_ref[...] = m2
    onehot3 = (jax.lax.broadcasted_iota(jnp.int32, (1, 128), 1) == 3
               ).astype(jnp.float32)
    tr_ref[...] = cdf * s + onehot3


def _edge_mlp(gat, ea, w0e, wrad, w1, b1, wc0, bc0, wc1):
    full = lambda r, c: pl.BlockSpec((r, c), lambda i: (0, 0))
    return pl.pallas_call(
        _edge_mlp_body,
        grid=(E_TOT // BE,),
        in_specs=[
            pl.BlockSpec((BE, HID + 128), lambda i: (i, 0)),
            pl.BlockSpec((BE, 16), lambda i: (i, 0)),
            full(16, HID), full(1, HID),
            full(HID, HID), full(1, HID),
            full(HID, HID), full(1, HID),
            full(1, HID),
        ],
        out_specs=[
            pl.BlockSpec((BE, HID), lambda i: (i, 0)),
            pl.BlockSpec((BE, 128), lambda i: (i, 0)),
        ],
        out_shape=[
            jax.ShapeDtypeStruct((E_TOT, HID), jnp.float32),
            jax.ShapeDtypeStruct((E_TOT, 128), jnp.float32),
        ],
    )(gat, ea, w0e, wrad.reshape(1, HID), w1, b1.reshape(1, HID),
      wc0, bc0.reshape(1, HID), wc1.reshape(1, HID))


def _node_mlp_body(h_ref, agg_ref, h0_ref, tr_ref, co_ref,
                   wh_ref, wa_ref, wx_ref, bn0_ref, wn1_ref, bn1_ref,
                   hout_ref, cout_ref):
    h = h_ref[...]
    acc = (jnp.dot(h, wh_ref[...], preferred_element_type=jnp.float32)
           + jnp.dot(agg_ref[...], wa_ref[...], preferred_element_type=jnp.float32)
           + jnp.dot(h0_ref[...], wx_ref[...], preferred_element_type=jnp.float32)
           + bn0_ref[...])
    o = jax.nn.silu(acc)
    hout_ref[...] = h + jnp.dot(o, wn1_ref[...],
                                preferred_element_type=jnp.float32) + bn1_ref[...]
    lane = jax.lax.broadcasted_iota(jnp.int32, (1, 128), 1)
    onehot3 = (lane == 3).astype(jnp.float32)
    diffmask = (lane < 3).astype(jnp.float32)
    tr = tr_ref[...]
    cnt = jnp.clip(jnp.sum(tr * onehot3, axis=1, keepdims=True), 1.0, None)
    cout_ref[...] = co_ref[...] + (tr * diffmask) / cnt


def _node_mlp(h, agg, h0p, trsum, coord, wh, wa, wx, bn0, wn1, bn1):
    full = lambda r, c: pl.BlockSpec((r, c), lambda i: (0, 0))
    return pl.pallas_call(
        _node_mlp_body,
        grid=(N_PAD // BN,),
        in_specs=[
            pl.BlockSpec((BN, HID), lambda i: (i, 0)),
            pl.BlockSpec((BN, HID), lambda i: (i, 0)),
            pl.BlockSpec((BN, 128), lambda i: (i, 0)),
            pl.BlockSpec((BN, 128), lambda i: (i, 0)),
            pl.BlockSpec((BN, 128), lambda i: (i, 0)),
            full(HID, HID), full(HID, HID), full(128, HID), full(1, HID),
            full(HID, HID), full(1, HID),
        ],
        out_specs=[
            pl.BlockSpec((BN, HID), lambda i: (i, 0)),
            pl.BlockSpec((BN, 128), lambda i: (i, 0)),
        ],
        out_shape=[
            jax.ShapeDtypeStruct((N_PAD, HID), jnp.float32),
            jax.ShapeDtypeStruct((N_PAD, 128), jnp.float32),
        ],
    )(h, agg, h0p, trsum, coord, wh, wa, wx, bn0.reshape(1, HID),
      wn1, bn1.reshape(1, HID))


_SC_MESH = plsc.VectorSubcoreMesh(core_axis_name="c", subcore_axis_name="s")


@functools.partial(
    pl.kernel,
    mesh=_SC_MESH,
    out_type=jax.ShapeDtypeStruct((E_TOT, HID + 128), jnp.float32),
    scratch_types=[
        pltpu.VMEM((2, CG), jnp.int32),
        pltpu.VMEM((2, CG), jnp.int32),
        pltpu.VMEM((2, CG, HID + 128), jnp.float32),
        pltpu.VMEM((2, CG, HID + 128), jnp.float32),
        pltpu.VMEM((2, CG, HID + 128), jnp.float32),
        pltpu.SemaphoreType.DMA,
        pltpu.SemaphoreType.DMA,
        pltpu.SemaphoreType.DMA,
        pltpu.SemaphoreType.DMA,
    ],
)
def _gather_sc(tabr, tabc, rows, cols, out,
               idxr, idxc, bufa, bufb, bufo,
               semg0, semg1, semo0, semo1):
    wid = lax.axis_index("s") * NC + lax.axis_index("c")
    base = wid * EW
    semg = (semg0, semg1)
    semo = (semo0, semo1)
    zero = jnp.zeros((16,), jnp.float32)

    def zrow(i, c):
        for u in range(1, 8):
            bufo[0, i, pl.ds(HID + u * 16, 16)] = zero
            bufo[1, i, pl.ds(HID + u * 16, 16)] = zero
        return c

    lax.fori_loop(0, CG, zrow, 0)

    def issue(k, par):
        off = base + k * CG
        pltpu.sync_copy(rows.at[pl.ds(off, CG)], idxr.at[par])
        pltpu.sync_copy(cols.at[pl.ds(off, CG)], idxc.at[par])
        pltpu.async_copy(tabr.at[idxr.at[par]], bufa.at[par], semg[par])
        pltpu.async_copy(tabc.at[idxc.at[par]], bufb.at[par], semg[par])

    issue(0, 0)
    issue(1, 1)

    def wait_store(k, par):
        off = base + k * CG
        pltpu.make_async_copy(
            bufo.at[par], out.at[pl.ds(off, CG)], semo[par]
        ).wait()

    def process(k, par):
        off = base + k * CG
        pltpu.make_async_copy(tabr.at[idxr.at[par]], bufa.at[par], semg[par]).wait()
        pltpu.make_async_copy(tabc.at[idxc.at[par]], bufb.at[par], semg[par]).wait()

        @pl.when(k >= 2)
        def _():
            wait_store(k - 2, par)

        def addrow(i, c2):
            for u in range(HID // 16):
                sl = pl.ds(u * 16, 16)
                bufo[par, i, sl] = bufa[par, i, sl] + bufb[par, i, sl]
            scd = pl.ds(HID, 16)
            bufo[par, i, scd] = bufa[par, i, scd] - bufb[par, i, scd]
            return c2

        lax.fori_loop(0, CG, addrow, 0)
        pltpu.async_copy(bufo.at[par], out.at[pl.ds(off, CG)], semo[par])

        @pl.when(k + 2 < NCHG)
        def _():
            issue(k + 2, par)

    def pair(k2, carry):
        process(2 * k2, 0)
        process(2 * k2 + 1, 1)
        return carry

    lax.fori_loop(0, NCHG // 2, pair, 0)
    wait_store(NCHG - 2, 0)
    wait_store(NCHG - 1, 1)


@functools.partial(
    pl.kernel,
    mesh=_SC_MESH,
    out_type=[
        jax.ShapeDtypeStruct((N_PAD, HID), jnp.float32),
        jax.ShapeDtypeStruct((N_PAD, 128), jnp.float32),
    ],
    scratch_types=[
        pltpu.VMEM((2, CS, HID), jnp.float32),
        pltpu.VMEM((2, CS, 128), jnp.float32),
        pltpu.VMEM((2, CS + 16), jnp.int32),
        pltpu.VMEM((2 * NPART + 16,), jnp.int32),
        pltpu.VMEM((NPP, HID), jnp.float32),
        pltpu.VMEM((NPP, 128), jnp.float32),
        pltpu.SemaphoreType.DMA,
        pltpu.SemaphoreType.DMA,
    ],
)
def _scatter_sc(m2, tr, rows, kbounds, agg, trsum,
                bufm, buft, idxv, kbv, acc, tacc, semm0, semm1):
    wid = lax.axis_index("s") * NC + lax.axis_index("c")
    semm = (semm0, semm1)
    zero = jnp.zeros((16,), jnp.float32)
    pltpu.sync_copy(kbounds, kbv.at[pl.ds(0, 2 * NPART)])

    for r in range(2):
        part = 2 * wid + r
        nbase = part * NPP

        def zrow(i, c):
            for u in range(HID // 16):
                acc[i, pl.ds(u * 16, 16)] = zero
            for u in range(8):
                tacc[i, pl.ds(u * 16, 16)] = zero
            return c

        lax.fori_loop(0, NPP, zrow, 0)

        kb = kbv[pl.ds(2 * part, 16)]
        k0 = kb[0]
        k1 = kb[1]

        def issue(k, par):
            @pl.when(k < k1)
            def _():
                off = k * CS
                pltpu.sync_copy(rows.at[pl.ds(off, CS)],
                                idxv.at[par, pl.ds(0, CS)])
                pltpu.async_copy(m2.at[pl.ds(off, CS)], bufm.at[par], semm[par])
                pltpu.async_copy(tr.at[pl.ds(off, CS)], buft.at[par], semm[par])

        issue(k0, 0)
        issue(k0 + 1, 1)

        def process(k, par):
            @pl.when(k < k1)
            def _():
                off = k * CS
                pltpu.make_async_copy(m2.at[pl.ds(off, CS)], bufm.at[par],
                                      semm[par]).wait()
                pltpu.make_async_copy(tr.at[pl.ds(off, CS)], buft.at[par],
                                      semm[par]).wait()

                def edge(i, c2):
                    loc = idxv[par, pl.ds(i, 16)][0] - nbase

                    @pl.when((loc >= 0) & (loc < NPP))
                    def _():
                        for u in range(HID // 16):
                            sl = pl.ds(u * 16, 16)
                            plsc.addupdate(acc.at[loc, sl], bufm[par, i, sl])
                        s16 = pl.ds(0, 16)
                        plsc.addupdate(tacc.at[loc, s16], buft[par, i, s16])

                    return c2

                lax.fori_loop(0, CS, edge, 0)
                issue(k + 2, par)

        def pair(j, carry):
            process(k0 + 2 * j, 0)
            process(k0 + 2 * j + 1, 1)
            return carry

        lax.fori_loop(0, (k1 - k0 + 1) // 2, pair, 0)
        pltpu.sync_copy(acc, agg.at[pl.ds(nbase, NPP)])
        pltpu.sync_copy(tacc, trsum.at[pl.ds(nbase, NPP)])


def kernel(h0, x, edge_index, edge_attr, params):
    N = h0.shape[0]
    row, col = edge_index[0], edge_index[1]
    perm = jnp.argsort(row)
    row_s, col_s, ea_s = row[perm], col[perm], edge_attr[perm]

    pnodes = jnp.arange(NPART + 1, dtype=jnp.int32) * NPP
    ebnd = jnp.searchsorted(row_s, pnodes, side="left").astype(jnp.int32)
    k0s = ebnd[:-1] // CS
    k1s = (ebnd[1:] + CS - 1) // CS
    kbounds = jnp.stack([k0s, k1s], axis=1).reshape(-1).astype(jnp.int32)

    h0p = jnp.zeros((N_PAD, 128), jnp.float32).at[:N].set(h0)
    coord = jnp.zeros((N_PAD, 128), jnp.float32).at[:N, :3].set(x)

    emb = params["embedding"]
    h = _embed(h0p, emb["w"], emb["b"])

    for lp in params["layers"]:
        W0 = lp["edge_mlp_0"]["w"]
        W0r, W0c = W0[:HID], W0[HID:2 * HID]
        wrad, W0e = W0[2 * HID], W0[2 * HID + 1:]
        tab_r, tab_c = _nodeproj(h, coord, W0r, W0c, lp["edge_mlp_0"]["b"])
        gat = _gather_sc(tab_r, tab_c, row_s, col_s)
        m2, trs = _edge_mlp(gat, ea_s, W0e, wrad,
                            lp["edge_mlp_1"]["w"], lp["edge_mlp_1"]["b"],
                            lp["coord_mlp_0"]["w"], lp["coord_mlp_0"]["b"],
                            lp["coord_mlp_1"]["w"][:, 0])
        agg, trsum = _scatter_sc(m2, trs, row_s, kbounds)
        h, coord = _node_mlp(h, agg, h0p, trsum, coord,
                             lp["node_mlp_0"]["w"][:HID],
                             lp["node_mlp_0"]["w"][HID:2 * HID],
                             lp["node_mlp_0"]["w"][2 * HID:],
                             lp["node_mlp_0"]["b"],
                             lp["node_mlp_1"]["w"], lp["node_mlp_1"]["b"])

    return h[:N], coord[:N, :3]


# speedup vs baseline: 1.1127x; 1.1127x over previous
import functools
import jax
import jax.numpy as jnp
from jax import lax
from jax.experimental import pallas as pl
from jax.experimental.pallas import tpu as pltpu
from jax.experimental.pallas import tpu_sc as plsc

N_PAD = 10240
HID = 256
BN = 2048
BE = 2000

NC, NS = 2, 16
NW = NC * NS
E_TOT = 320000
EW = E_TOT // NW
CG = 40
NCHG = EW // CG
CS = 32
NPART = 2 * NW
NPP = N_PAD // NPART


def _embed_body(h0_ref, w_ref, b_ref, out_ref):
    out_ref[...] = jnp.dot(h0_ref[...], w_ref[...],
                           preferred_element_type=jnp.float32) + b_ref[...]


def _embed(h0p, w, b):
    return pl.pallas_call(
        _embed_body,
        grid=(N_PAD // BN,),
        in_specs=[
            pl.BlockSpec((BN, 128), lambda i: (i, 0)),
            pl.BlockSpec((128, HID), lambda i: (0, 0)),
            pl.BlockSpec((1, HID), lambda i: (0, 0)),
        ],
        out_specs=pl.BlockSpec((BN, HID), lambda i: (i, 0)),
        out_shape=jax.ShapeDtypeStruct((N_PAD, HID), jnp.float32),
    )(h0p, w, b.reshape(1, HID))


def _pack_bf16(x):
    hi = lax.bitcast_convert_type(x[:, :128], jnp.uint32)
    lo = lax.bitcast_convert_type(x[:, 128:], jnp.uint32)
    hi = (hi + jnp.uint32(0x8000)) & jnp.uint32(0xFFFF0000)
    lo = (lo + jnp.uint32(0x8000)) >> 16
    return lax.bitcast_convert_type(hi | lo, jnp.int32)


def _unpack_bf16(u):
    b = lax.bitcast_convert_type(u, jnp.uint32)
    hi = lax.bitcast_convert_type(b & jnp.uint32(0xFFFF0000), jnp.float32)
    lo = lax.bitcast_convert_type(b << 16, jnp.float32)
    return hi, lo


def _nodeproj_body(h_ref, wr_ref, wc_ref, b0_ref, prer_ref, prec_ref):
    h = h_ref[...]
    prer_ref[...] = _pack_bf16(
        jnp.dot(h, wr_ref[...], preferred_element_type=jnp.float32)
        + b0_ref[...])
    prec_ref[...] = _pack_bf16(
        jnp.dot(h, wc_ref[...], preferred_element_type=jnp.float32))


def _nodeproj(h, wr, wc, b0):
    return pl.pallas_call(
        _nodeproj_body,
        grid=(N_PAD // BN,),
        in_specs=[
            pl.BlockSpec((BN, HID), lambda i: (i, 0)),
            pl.BlockSpec((HID, HID), lambda i: (0, 0)),
            pl.BlockSpec((HID, HID), lambda i: (0, 0)),
            pl.BlockSpec((1, HID), lambda i: (0, 0)),
        ],
        out_specs=[
            pl.BlockSpec((BN, 128), lambda i: (i, 0)),
            pl.BlockSpec((BN, 128), lambda i: (i, 0)),
        ],
        out_shape=[
            jax.ShapeDtypeStruct((N_PAD, 128), jnp.int32),
            jax.ShapeDtypeStruct((N_PAD, 128), jnp.int32),
        ],
    )(h, wr, wc, b0.reshape(1, HID))


def _edge_mlp_body(pr_ref, pc_ref, cd_ref, ea_ref, w0e_ref, wrad_ref,
                   w1_ref, b1_ref, wc0_ref, bc0_ref, wc1_ref,
                   m2_ref, tr_ref):
    cdf = cd_ref[...]
    radial = jnp.sum(cdf * cdf, axis=1, keepdims=True)
    rhi, rlo = _unpack_bf16(pr_ref[...])
    chi, clo = _unpack_bf16(pc_ref[...])
    pre = (jnp.concatenate([rhi + chi, rlo + clo], axis=1)
           + radial * wrad_ref[...]
           + jnp.dot(ea_ref[...], w0e_ref[...],
                     preferred_element_type=jnp.float32))
    m = jax.nn.silu(pre)
    m2 = jax.nn.silu(jnp.dot(m, w1_ref[...],
                             preferred_element_type=jnp.float32) + b1_ref[...])
    p = jax.nn.silu(jnp.dot(m2, wc0_ref[...],
                            preferred_element_type=jnp.float32) + bc0_ref[...])
    s = jnp.sum(p * wc1_ref[...], axis=1, keepdims=True)
    m2_ref[...] = m2
    onehot3 = (jax.lax.broadcasted_iota(jnp.int32, (1, 128), 1) == 3
               ).astype(jnp.float32)
    tr_ref[...] = cdf * s + onehot3


def _edge_mlp(prg, pcg, cdf, ea, w0e, wrad, w1, b1, wc0, bc0, wc1):
    full = lambda r, c: pl.BlockSpec((r, c), lambda i: (0, 0))
    return pl.pallas_call(
        _edge_mlp_body,
        grid=(E_TOT // BE,),
        in_specs=[
            pl.BlockSpec((BE, 128), lambda i: (i, 0)),
            pl.BlockSpec((BE, 128), lambda i: (i, 0)),
            pl.BlockSpec((BE, 128), lambda i: (i, 0)),
            pl.BlockSpec((BE, 16), lambda i: (i, 0)),
            full(16, HID), full(1, HID),
            full(HID, HID), full(1, HID),
            full(HID, HID), full(1, HID),
            full(1, HID),
        ],
        out_specs=[
            pl.BlockSpec((BE, HID), lambda i: (i, 0)),
            pl.BlockSpec((BE, 128), lambda i: (i, 0)),
        ],
        out_shape=[
            jax.ShapeDtypeStruct((E_TOT, HID), jnp.float32),
            jax.ShapeDtypeStruct((E_TOT, 128), jnp.float32),
        ],
    )(prg, pcg, cdf, ea, w0e, wrad.reshape(1, HID), w1, b1.reshape(1, HID),
      wc0, bc0.reshape(1, HID), wc1.reshape(1, HID))


def _node_mlp_body(h_ref, agg_ref, h0_ref, tr_ref, co_ref,
                   wh_ref, wa_ref, wx_ref, bn0_ref, wn1_ref, bn1_ref,
                   hout_ref, cout_ref):
    h = h_ref[...]
    acc = (jnp.dot(h, wh_ref[...], preferred_element_type=jnp.float32)
           + jnp.dot(agg_ref[...], wa_ref[...], preferred_element_type=jnp.float32)
           + jnp.dot(h0_ref[...], wx_ref[...], preferred_element_type=jnp.float32)
           + bn0_ref[...])
    o = jax.nn.silu(acc)
    hout_ref[...] = h + jnp.dot(o, wn1_ref[...],
                                preferred_element_type=jnp.float32) + bn1_ref[...]
    lane = jax.lax.broadcasted_iota(jnp.int32, (1, 128), 1)
    onehot3 = (lane == 3).astype(jnp.float32)
    diffmask = (lane < 3).astype(jnp.float32)
    tr = tr_ref[...]
    cnt = jnp.clip(jnp.sum(tr * onehot3, axis=1, keepdims=True), 1.0, None)
    cout_ref[...] = co_ref[...] + (tr * diffmask) / cnt


def _node_mlp(h, agg, h0p, trsum, coord, wh, wa, wx, bn0, wn1, bn1):
    full = lambda r, c: pl.BlockSpec((r, c), lambda i: (0, 0))
    return pl.pallas_call(
        _node_mlp_body,
        grid=(N_PAD // BN,),
        in_specs=[
            pl.BlockSpec((BN, HID), lambda i: (i, 0)),
            pl.BlockSpec((BN, HID), lambda i: (i, 0)),
            pl.BlockSpec((BN, 128), lambda i: (i, 0)),
            pl.BlockSpec((BN, 128), lambda i: (i, 0)),
            pl.BlockSpec((BN, 128), lambda i: (i, 0)),
            full(HID, HID), full(HID, HID), full(128, HID), full(1, HID),
            full(HID, HID), full(1, HID),
        ],
        out_specs=[
            pl.BlockSpec((BN, HID), lambda i: (i, 0)),
            pl.BlockSpec((BN, 128), lambda i: (i, 0)),
        ],
        out_shape=[
            jax.ShapeDtypeStruct((N_PAD, HID), jnp.float32),
            jax.ShapeDtypeStruct((N_PAD, 128), jnp.float32),
        ],
    )(h, agg, h0p, trsum, coord, wh, wa, wx, bn0.reshape(1, HID),
      wn1, bn1.reshape(1, HID))


_SC_MESH = plsc.VectorSubcoreMesh(core_axis_name="c", subcore_axis_name="s")


@functools.partial(
    pl.kernel,
    mesh=_SC_MESH,
    out_type=[
        jax.ShapeDtypeStruct((E_TOT, 128), jnp.int32),
        jax.ShapeDtypeStruct((E_TOT, 128), jnp.int32),
        jax.ShapeDtypeStruct((E_TOT, 128), jnp.float32),
    ],
    scratch_types=[
        pltpu.VMEM((4, CG), jnp.int32),
        pltpu.VMEM((4, CG), jnp.int32),
        pltpu.VMEM((4, CG, 128), jnp.int32),
        pltpu.VMEM((4, CG, 128), jnp.int32),
        pltpu.VMEM((4, CG, 128), jnp.float32),
        pltpu.VMEM((4, CG, 128), jnp.float32),
        pltpu.VMEM((4, CG, 128), jnp.float32),
        pltpu.SemaphoreType.DMA,
        pltpu.SemaphoreType.DMA,
        pltpu.SemaphoreType.DMA,
        pltpu.SemaphoreType.DMA,
        pltpu.SemaphoreType.DMA,
        pltpu.SemaphoreType.DMA,
        pltpu.SemaphoreType.DMA,
        pltpu.SemaphoreType.DMA,
    ],
)
def _gather_sc(prer, prec, coord, rows, cols, outr, outc, outd,
               idxr, idxc, bufa, bufb, bufc, bufd, bufcd,
               semg0, semg1, semg2, semg3, semo0, semo1, semo2, semo3):
    wid = lax.axis_index("s") * NC + lax.axis_index("c")
    base = wid * EW
    semg = (semg0, semg1, semg2, semg3)
    semo = (semo0, semo1, semo2, semo3)
    zero = jnp.zeros((16,), jnp.float32)

    def zrow(i, c):
        for par in range(4):
            for u in range(1, 8):
                bufcd[par, i, pl.ds(u * 16, 16)] = zero
        return c

    lax.fori_loop(0, CG, zrow, 0)

    def issue(k, par):
        off = base + k * CG
        pltpu.sync_copy(rows.at[pl.ds(off, CG)], idxr.at[par])
        pltpu.sync_copy(cols.at[pl.ds(off, CG)], idxc.at[par])
        pltpu.async_copy(prer.at[idxr.at[par]], bufa.at[par], semg[par])
        pltpu.async_copy(prec.at[idxc.at[par]], bufb.at[par], semg[par])
        pltpu.async_copy(coord.at[idxr.at[par]], bufc.at[par], semg[par])
        pltpu.async_copy(coord.at[idxc.at[par]], bufd.at[par], semg[par])

    issue(0, 0)
    issue(1, 1)

    def wait_store(k, par):
        off = base + k * CG
        pltpu.make_async_copy(
            bufa.at[par], outr.at[pl.ds(off, CG)], semo[par]).wait()
        pltpu.make_async_copy(
            bufb.at[par], outc.at[pl.ds(off, CG)], semo[par]).wait()
        pltpu.make_async_copy(
            bufcd.at[par], outd.at[pl.ds(off, CG)], semo[par]).wait()

    def process(k, par):
        off = base + k * CG
        pltpu.make_async_copy(prer.at[idxr.at[par]], bufa.at[par], semg[par]).wait()
        pltpu.make_async_copy(prec.at[idxc.at[par]], bufb.at[par], semg[par]).wait()
        pltpu.make_async_copy(coord.at[idxr.at[par]], bufc.at[par], semg[par]).wait()
        pltpu.make_async_copy(coord.at[idxc.at[par]], bufd.at[par], semg[par]).wait()

        @pl.when(k >= 2)
        def _():
            wait_store(k - 2, (par + 2) % 4)

        def subrow(i, c2):
            s16 = pl.ds(0, 16)
            bufcd[par, i, s16] = bufc[par, i, s16] - bufd[par, i, s16]
            return c2

        lax.fori_loop(0, CG, subrow, 0)
        pltpu.async_copy(bufa.at[par], outr.at[pl.ds(off, CG)], semo[par])
        pltpu.async_copy(bufb.at[par], outc.at[pl.ds(off, CG)], semo[par])
        pltpu.async_copy(bufcd.at[par], outd.at[pl.ds(off, CG)], semo[par])

        @pl.when(k + 2 < NCHG)
        def _():
            issue(k + 2, (par + 2) % 4)

    def quad(k4, carry):
        process(4 * k4, 0)
        process(4 * k4 + 1, 1)
        process(4 * k4 + 2, 2)
        process(4 * k4 + 3, 3)
        return carry

    lax.fori_loop(0, NCHG // 4, quad, 0)
    for k in range(NCHG - NCHG % 4, NCHG):
        process(k, k % 4)
    wait_store(NCHG - 2, (NCHG - 2) % 4)
    wait_store(NCHG - 1, (NCHG - 1) % 4)


@functools.partial(
    pl.kernel,
    mesh=_SC_MESH,
    out_type=[
        jax.ShapeDtypeStruct((N_PAD, HID), jnp.float32),
        jax.ShapeDtypeStruct((N_PAD, 128), jnp.float32),
    ],
    scratch_types=[
        pltpu.VMEM((2, CS, HID), jnp.float32),
        pltpu.VMEM((2, CS, 128), jnp.float32),
        pltpu.VMEM((2, CS + 16), jnp.int32),
        pltpu.VMEM((2 * NPART + 16,), jnp.int32),
        pltpu.VMEM((NPP, HID), jnp.float32),
        pltpu.VMEM((NPP, 128), jnp.float32),
        pltpu.SemaphoreType.DMA,
        pltpu.SemaphoreType.DMA,
    ],
)
def _scatter_sc(m2, tr, rows, kbounds, agg, trsum,
                bufm, buft, idxv, kbv, acc, tacc, semm0, semm1):
    wid = lax.axis_index("s") * NC + lax.axis_index("c")
    semm = (semm0, semm1)
    zero = jnp.zeros((16,), jnp.float32)
    pltpu.sync_copy(kbounds, kbv.at[pl.ds(0, 2 * NPART)])

    for r in range(2):
        part = 2 * wid + r
        nbase = part * NPP

        def zrow(i, c):
            for u in range(HID // 16):
                acc[i, pl.ds(u * 16, 16)] = zero
            for u in range(8):
                tacc[i, pl.ds(u * 16, 16)] = zero
            return c

        lax.fori_loop(0, NPP, zrow, 0)

        kb = kbv[pl.ds(2 * part, 16)]
        k0 = kb[0]
        k1 = kb[1]

        def issue(k, par):
            @pl.when(k < k1)
            def _():
                off = k * CS
                pltpu.sync_copy(rows.at[pl.ds(off, CS)],
                                idxv.at[par, pl.ds(0, CS)])
                pltpu.async_copy(m2.at[pl.ds(off, CS)], bufm.at[par], semm[par])
                pltpu.async_copy(tr.at[pl.ds(off, CS)], buft.at[par], semm[par])

        issue(k0, 0)
        issue(k0 + 1, 1)

        def process(k, par):
            @pl.when(k < k1)
            def _():
                off = k * CS
                pltpu.make_async_copy(m2.at[pl.ds(off, CS)], bufm.at[par],
                                      semm[par]).wait()
                pltpu.make_async_copy(tr.at[pl.ds(off, CS)], buft.at[par],
                                      semm[par]).wait()

                def edge(i, c2):
                    loc = idxv[par, pl.ds(i, 16)][0] - nbase

                    @pl.when((loc >= 0) & (loc < NPP))
                    def _():
                        for u in range(HID // 16):
                            sl = pl.ds(u * 16, 16)
                            plsc.addupdate(acc.at[loc, sl], bufm[par, i, sl])
                        s16 = pl.ds(0, 16)
                        plsc.addupdate(tacc.at[loc, s16], buft[par, i, s16])

                    return c2

                lax.fori_loop(0, CS, edge, 0)
                issue(k + 2, par)

        def pair(j, carry):
            process(k0 + 2 * j, 0)
            process(k0 + 2 * j + 1, 1)
            return carry

        lax.fori_loop(0, (k1 - k0 + 1) // 2, pair, 0)
        pltpu.sync_copy(acc, agg.at[pl.ds(nbase, NPP)])
        pltpu.sync_copy(tacc, trsum.at[pl.ds(nbase, NPP)])


def kernel(h0, x, edge_index, edge_attr, params):
    N = h0.shape[0]
    row, col = edge_index[0], edge_index[1]
    perm = jnp.argsort(row)
    row_s, col_s, ea_s = row[perm], col[perm], edge_attr[perm]

    pnodes = jnp.arange(NPART + 1, dtype=jnp.int32) * NPP
    ebnd = jnp.searchsorted(row_s, pnodes, side="left").astype(jnp.int32)
    k0s = ebnd[:-1] // CS
    k1s = (ebnd[1:] + CS - 1) // CS
    kbounds = jnp.stack([k0s, k1s], axis=1).reshape(-1).astype(jnp.int32)

    h0p = jnp.zeros((N_PAD, 128), jnp.float32).at[:N].set(h0)
    coord = jnp.zeros((N_PAD, 128), jnp.float32).at[:N, :3].set(x)

    emb = params["embedding"]
    h = _embed(h0p, emb["w"], emb["b"])

    for lp in params["layers"]:
        W0 = lp["edge_mlp_0"]["w"]
        W0r, W0c = W0[:HID], W0[HID:2 * HID]
        wrad, W0e = W0[2 * HID], W0[2 * HID + 1:]
        pre_r, pre_c = _nodeproj(h, W0r, W0c, lp["edge_mlp_0"]["b"])
        prg, pcg, cdf = _gather_sc(pre_r, pre_c, coord, row_s, col_s)
        m2, trs = _edge_mlp(prg, pcg, cdf, ea_s, W0e, wrad,
                            lp["edge_mlp_1"]["w"], lp["edge_mlp_1"]["b"],
                            lp["coord_mlp_0"]["w"], lp["coord_mlp_0"]["b"],
                            lp["coord_mlp_1"]["w"][:, 0])
        agg, trsum = _scatter_sc(m2, trs, row_s, kbounds)
        h, coord = _node_mlp(h, agg, h0p, trsum, coord,
                             lp["node_mlp_0"]["w"][:HID],
                             lp["node_mlp_0"]["w"][HID:2 * HID],
                             lp["node_mlp_0"]["w"][2 * HID:],
                             lp["node_mlp_0"]["b"],
                             lp["node_mlp_1"]["w"], lp["node_mlp_1"]["b"])

    return h[:N], coord[:N, :3]
